# baseline (speedup 1.0000x reference)
import functools

import jax
import jax.numpy as jnp
from jax import lax
from jax.experimental import pallas as pl
from jax.experimental.pallas import tpu as pltpu
from jax.experimental.pallas import tpu_sc as plsc

_B = 4096
_N = 8192
_ROWS_PER_BLOCK = 512
_R_SC = 512
_R_TC = _B - _R_SC

_SC_PARAMS = pltpu.CompilerParams(needs_layout_passes=False)


def _argmin_body(dist_ref, out_ref):
    d = dist_ref[...]
    m = jnp.min(d, axis=1, keepdims=True)
    ii = lax.broadcasted_iota(jnp.int32, d.shape, 1)
    cand = jnp.where(d == m, ii, jnp.int32(_N))
    out_ref[...] = jnp.min(cand, axis=1)


def _argmin_indices(distances):
    nb = _R_TC // _ROWS_PER_BLOCK
    return pl.pallas_call(
        _argmin_body,
        grid=(nb,),
        in_specs=[pl.BlockSpec((_ROWS_PER_BLOCK, _N), lambda i: (i, 0))],
        out_specs=pl.BlockSpec((_ROWS_PER_BLOCK,), lambda i: (i,)),
        out_shape=jax.ShapeDtypeStruct((_R_TC,), jnp.int32),
    )(distances)


def _sc_gather(reg_vals, win_idx):
    info = plsc.get_sparse_core_info()
    nc, ns, lanes = info.num_cores, info.num_subcores, info.num_lanes
    nw = nc * ns
    bpw = _R_TC // nw
    nch = bpw // lanes
    mesh = plsc.VectorSubcoreMesh(core_axis_name="c", subcore_axis_name="s")

    @functools.partial(
        pl.kernel,
        mesh=mesh,
        out_type=jax.ShapeDtypeStruct((_R_TC,), jnp.float32),
        compiler_params=_SC_PARAMS,
        scratch_types=[
            pltpu.VMEM((bpw,), jnp.int32),
            pltpu.VMEM((bpw * 8,), jnp.float32),
            pltpu.VMEM((bpw,), jnp.float32),
            pltpu.SemaphoreType.DMA,
        ],
    )
    def gather_kernel(tab_hbm, idx_hbm, out_hbm, idx_v, buf_v, o_v, sem):
        wid = lax.axis_index("s") * nc + lax.axis_index("c")
        base = wid * bpw
        pltpu.sync_copy(idx_hbm.at[pl.ds(base, bpw)], idx_v)
        copies = []
        for c in range(nch):
            j16 = idx_v[pl.ds(c * lanes, lanes)]
            for m in range(lanes):
                k = c * lanes + m
                col0 = lax.div(j16[m], 8) * 8
                copies.append(pltpu.make_async_copy(
                    tab_hbm.at[base + k, pl.ds(col0, 8)],
                    buf_v.at[pl.ds(k * 8, 8)],
                    sem,
                ))
        for cp in copies:
            cp.start()
        for cp in copies:
            cp.wait()
        for c in range(nch):
            j = idx_v[pl.ds(c * lanes, lanes)]
            loc = (lax.iota(jnp.int32, lanes) + (c * lanes)) * 8 + lax.rem(
                j, 8)
            o_v[pl.ds(c * lanes, lanes)] = plsc.load_gather(buf_v, [loc])
        pltpu.sync_copy(o_v, out_hbm.at[pl.ds(base, bpw)])

    return gather_kernel(reg_vals, win_idx)


def _sc_argmin_gather(reg_vals, distances):
    info = plsc.get_sparse_core_info()
    nc, ns, lanes = info.num_cores, info.num_subcores, info.num_lanes
    nw = nc * ns
    rpw = _R_SC // nw
    unroll = 8
    nit = _N // lanes // unroll
    mesh = plsc.VectorSubcoreMesh(core_axis_name="c", subcore_axis_name="s")

    @functools.partial(
        pl.kernel,
        mesh=mesh,
        out_type=jax.ShapeDtypeStruct((_R_SC,), jnp.float32),
        compiler_params=_SC_PARAMS,
        scratch_types=[
            pltpu.VMEM((2 * _N,), jnp.float32),
            pltpu.VMEM((rpw * 8,), jnp.float32),
            pltpu.VMEM((rpw,), jnp.int32),
            pltpu.VMEM((rpw,), jnp.float32),
            pltpu.SemaphoreType.DMA,
            pltpu.SemaphoreType.DMA,
            pltpu.SemaphoreType.DMA,
        ],
    )
    def k(tab_hbm, dist_hbm, out_hbm, buf_v, gbuf_v, win_v, o_v,
          sem0, sem1, semg):
        wid = lax.axis_index("s") * nc + lax.axis_index("c")
        r0 = _R_TC + wid * rpw
        iota16 = lax.iota(jnp.int32, lanes)
        sems = (sem0, sem1)
        row_cp = [
            pltpu.make_async_copy(
                dist_hbm.at[r0 + r, :],
                buf_v.at[pl.ds((r % 2) * _N, _N)],
                sems[r % 2],
            )
            for r in range(rpw)
        ]
        row_cp[0].start()
        gcopies = []
        wacc = jnp.zeros((lanes,), jnp.int32)
        for r in range(rpw):
            if r + 1 < rpw:
                row_cp[r + 1].start()
            row_cp[r].wait()
            half = (r % 2) * _N

            def body(v, carry, half=half):
                vmin, vidx = carry
                for u in range(unroll):
                    pos = (v * unroll + u) * lanes
                    val = buf_v[pl.ds(half + pos, lanes)]
                    cur = iota16 + pos
                    upd = val < vmin
                    vmin = jnp.where(upd, val, vmin)
                    vidx = jnp.where(upd, cur, vidx)
                return vmin, vidx

            vmin0 = jnp.full((lanes,), jnp.inf, jnp.float32)
            vidx0 = jnp.zeros((lanes,), jnp.int32)
            vmin, vidx = lax.fori_loop(0, nit, body, (vmin0, vidx0))
            m = jnp.min(vmin)
            win = jnp.min(jnp.where(vmin == m, vidx, jnp.int32(_N)))
            col0 = lax.div(win, 8) * 8
            gcp = pltpu.make_async_copy(
                tab_hbm.at[r0 + r, pl.ds(col0, 8)],
                gbuf_v.at[pl.ds(r * 8, 8)],
                semg,
            )
            gcp.start()
            gcopies.append(gcp)
            wacc = jnp.where(iota16 == (r % lanes), jnp.full((lanes,), win),
                             wacc)
            if r % lanes == lanes - 1:
                win_v[pl.ds((r // lanes) * lanes, lanes)] = wacc
                wacc = jnp.zeros((lanes,), jnp.int32)
        for gcp in gcopies:
            gcp.wait()
        for c in range(rpw // lanes):
            j = win_v[pl.ds(c * lanes, lanes)]
            loc = (iota16 + c * lanes) * 8 + lax.rem(j, 8)
            o_v[pl.ds(c * lanes, lanes)] = plsc.load_gather(gbuf_v, [loc])
        pltpu.sync_copy(o_v, out_hbm.at[pl.ds(wid * rpw, rpw)])

    return k(reg_vals, distances)


def kernel(reg_vals, distances):
    preds_sc = _sc_argmin_gather(reg_vals, distances)
    win_idx = _argmin_indices(distances)
    preds_tc = _sc_gather(reg_vals, win_idx)
    return jnp.concatenate([preds_tc, preds_sc])


# speedup vs baseline: 1.2856x; 1.2856x over previous
import functools

import jax
import jax.numpy as jnp
from jax import lax
from jax.experimental import pallas as pl
from jax.experimental.pallas import tpu as pltpu
from jax.experimental.pallas import tpu_sc as plsc

_B = 4096
_N = 8192
_ROWS_PER_BLOCK = 512


def _argmin_body(dist_ref, out_ref):
    d = dist_ref[...]
    m = jnp.min(d, axis=1, keepdims=True)
    ii = lax.broadcasted_iota(jnp.int32, d.shape, 1)
    cand = jnp.where(d == m, ii, jnp.int32(_N))
    out_ref[...] = jnp.min(cand, axis=1)


def _argmin_indices(distances):
    nb = _B // _ROWS_PER_BLOCK
    return pl.pallas_call(
        _argmin_body,
        grid=(nb,),
        in_specs=[pl.BlockSpec((_ROWS_PER_BLOCK, _N), lambda i: (i, 0))],
        out_specs=pl.BlockSpec((_ROWS_PER_BLOCK,), lambda i: (i,)),
        out_shape=jax.ShapeDtypeStruct((_B,), jnp.int32),
    )(distances)


def _sc_gather(reg_vals, win_idx):
    info = plsc.get_sparse_core_info()
    nc, ns, lanes = info.num_cores, info.num_subcores, info.num_lanes
    nw = nc * ns
    bpw = _B // nw
    nch = bpw // lanes
    mesh = plsc.VectorSubcoreMesh(core_axis_name="c", subcore_axis_name="s")

    @functools.partial(
        pl.kernel,
        mesh=mesh,
        out_type=jax.ShapeDtypeStruct((_B,), jnp.float32),
        compiler_params=pltpu.CompilerParams(needs_layout_passes=False),
        scratch_types=[
            pltpu.VMEM((bpw,), jnp.int32),
            pltpu.VMEM((bpw * 8,), jnp.float32),
            pltpu.VMEM((bpw,), jnp.float32),
            pltpu.SemaphoreType.DMA,
        ],
    )
    def gather_kernel(tab_hbm, idx_hbm, out_hbm, idx_v, buf_v, o_v, sem):
        wid = lax.axis_index("s") * nc + lax.axis_index("c")
        base = wid * bpw
        pltpu.sync_copy(idx_hbm.at[pl.ds(base, bpw)], idx_v)
        copies = []
        for c in range(nch):
            j16 = idx_v[pl.ds(c * lanes, lanes)]
            for m in range(lanes):
                k = c * lanes + m
                col0 = lax.div(j16[m], 8) * 8
                copies.append(pltpu.make_async_copy(
                    tab_hbm.at[base + k, pl.ds(col0, 8)],
                    buf_v.at[pl.ds(k * 8, 8)],
                    sem,
                ))
        for cp in copies:
            cp.start()
        for cp in copies:
            cp.wait()
        for c in range(nch):
            j = idx_v[pl.ds(c * lanes, lanes)]
            loc = (lax.iota(jnp.int32, lanes) + (c * lanes)) * 8 + lax.rem(
                j, 8)
            o_v[pl.ds(c * lanes, lanes)] = plsc.load_gather(buf_v, [loc])
        pltpu.sync_copy(o_v, out_hbm.at[pl.ds(base, bpw)])

    return gather_kernel(reg_vals, win_idx)


def kernel(reg_vals, distances):
    win_idx = _argmin_indices(distances)
    return _sc_gather(reg_vals, win_idx)
